# initial kernel scaffold (unmeasured)
import jax
import jax.numpy as jnp
from jax import lax
from jax.experimental import pallas as pl
from jax.experimental.pallas import tpu as pltpu

N_DEV = 4
M_BLK = 1024
K = 4096
N = 8192
K_BLK = 1024
K_CHUNK = 512
CHUNKS_PER_BLK = K_BLK // K_CHUNK


def kernel(x, w_mat):
    def body(x_ref, w_hbm, out_ref, xb, xg, wstage, amax_mine, amax_buf,
             send_sems, recv_sems, am_send_sems, am_recv_sems, w_sems):
        my = lax.axis_index("i")

        barrier = pltpu.get_barrier_semaphore()
        for off in (1, 2, 3):
            dst = lax.rem(my + off, N_DEV)
            pl.semaphore_signal(
                barrier, inc=1,
                device_id=(dst,), device_id_type=pl.DeviceIdType.MESH,
            )
        pl.semaphore_wait(barrier, N_DEV - 1)

        amax_buf[...] = jnp.zeros_like(amax_buf)
        xb[...] = x_ref[...].astype(jnp.bfloat16)

        a2a_sends = []
        for off in (1, 2, 3):
            dst = lax.rem(my + off, N_DEV)
            rdma = pltpu.make_async_remote_copy(
                src_ref=xb.at[pl.ds(dst * M_BLK, M_BLK), :],
                dst_ref=xg.at[:, pl.ds(my * K_BLK, K_BLK)],
                send_sem=send_sems.at[off - 1],
                recv_sem=recv_sems.at[my],
                device_id=(dst,),
                device_id_type=pl.DeviceIdType.MESH,
            )
            rdma.start()
            a2a_sends.append(rdma)

        xg[:, pl.ds(my * K_BLK, K_BLK)] = xb[pl.ds(my * M_BLK, M_BLK), :]

        def recv_desc(src):
            return pltpu.make_async_remote_copy(
                src_ref=xb.at[pl.ds(0, M_BLK), :],
                dst_ref=xg.at[:, pl.ds(src * K_BLK, K_BLK)],
                send_sem=send_sems.at[0],
                recv_sem=recv_sems.at[src],
                device_id=(src,),
                device_id_type=pl.DeviceIdType.MESH,
            )

        block_order = [
            (0, my),
            (1, lax.rem(my + 1, N_DEV)),
            (3, lax.rem(my + 3, N_DEV)),
            (2, lax.rem(my + 2, N_DEV)),
        ]
        chunk_offs = []
        for _, j in block_order:
            for s in range(CHUNKS_PER_BLK):
                chunk_offs.append(j * K_BLK + s * K_CHUNK)

        def w_dma(c):
            return pltpu.make_async_copy(
                w_hbm.at[pl.ds(chunk_offs[c], K_CHUNK), :],
                wstage.at[c % 2],
                w_sems.at[c % 2],
            )

        w_dma(0).start()
        n_chunks = len(chunk_offs)
        for c in range(n_chunks):
            if c + 1 < n_chunks:
                w_dma(c + 1).start()
            blk = c // CHUNKS_PER_BLK
            if c % CHUNKS_PER_BLK == 0 and blk > 0:
                recv_desc(block_order[blk][1]).wait_recv()
            w_dma(c).wait()
            xg_k = xg[:, pl.ds(chunk_offs[c], K_CHUNK)]
            wb = wstage[c % 2].astype(jnp.bfloat16)
            part = jnp.dot(xg_k, wb, preferred_element_type=jnp.float32)
            if c == 0:
                out_ref[...] = part
            else:
                out_ref[...] += part

        local_amax = jnp.max(jnp.abs(out_ref[...]))
        amax_mine[...] = jnp.full((8, 128), local_amax, jnp.float32)
        am_sends = []
        for off in (1, 2, 3):
            dst = lax.rem(my + off, N_DEV)
            rdma = pltpu.make_async_remote_copy(
                src_ref=amax_mine,
                dst_ref=amax_buf.at[my],
                send_sem=am_send_sems.at[off - 1],
                recv_sem=am_recv_sems.at[my],
                device_id=(dst,),
                device_id_type=pl.DeviceIdType.MESH,
            )
            rdma.start()
            am_sends.append(rdma)
        for off in (1, 2, 3):
            src = lax.rem(my + off, N_DEV)
            pltpu.make_async_remote_copy(
                src_ref=amax_mine,
                dst_ref=amax_buf.at[src],
                send_sem=am_send_sems.at[0],
                recv_sem=am_recv_sems.at[src],
                device_id=(src,),
                device_id_type=pl.DeviceIdType.MESH,
            ).wait_recv()
        amax = jnp.maximum(jnp.max(amax_buf[...]), local_amax)

        scale = amax / 127.0
        q = jnp.clip(jnp.round(out_ref[...] / scale), -127.0, 127.0)
        out_ref[...] = q * scale

        for r in a2a_sends + am_sends:
            r.wait_send()

    return pl.pallas_call(
        body,
        out_shape=jax.ShapeDtypeStruct((M_BLK, N), jnp.float32),
        in_specs=[
            pl.BlockSpec(memory_space=pltpu.VMEM),
            pl.BlockSpec(memory_space=pltpu.ANY),
        ],
        out_specs=pl.BlockSpec(memory_space=pltpu.VMEM),
        scratch_shapes=[
            pltpu.VMEM((N_DEV * M_BLK, K_BLK), jnp.bfloat16),
            pltpu.VMEM((M_BLK, K), jnp.bfloat16),
            pltpu.VMEM((2, K_CHUNK, N), jnp.float32),
            pltpu.VMEM((8, 128), jnp.float32),
            pltpu.VMEM((N_DEV, 8, 128), jnp.float32),
            pltpu.SemaphoreType.DMA((3,)),
            pltpu.SemaphoreType.DMA((N_DEV,)),
            pltpu.SemaphoreType.DMA((3,)),
            pltpu.SemaphoreType.DMA((N_DEV,)),
            pltpu.SemaphoreType.DMA((2,)),
        ],
        compiler_params=pltpu.CompilerParams(collective_id=0),
    )(x, w_mat)


# baseline (device time: 196728 ns/iter reference)
import jax
import jax.numpy as jnp
from jax import lax
from jax.experimental import pallas as pl
from jax.experimental.pallas import tpu as pltpu

N_DEV = 4
M_BLK = 1024
K = 4096
N = 8192
K_BLK = 1024
K_CHUNK = 512
N_TILE = 512
X_ROWS = 512


def kernel(x, w_mat):
    def body(x_hbm, w_hbm, out_ref, xb, xg, xstage, wstage, amax_mine,
             amax_buf, send_sems, recv_sems, am_send_sems, am_recv_sems,
             x_sems, w_sems):
        my = lax.axis_index("i")

        barrier = pltpu.get_barrier_semaphore()
        for off in (1, 2, 3):
            dst = lax.rem(my + off, N_DEV)
            pl.semaphore_signal(
                barrier, inc=1,
                device_id=(dst,), device_id_type=pl.DeviceIdType.MESH,
            )
        pl.semaphore_wait(barrier, N_DEV - 1)

        amax_buf[...] = jnp.zeros_like(amax_buf)

        row_jobs = []
        for off in (2, 1, 3, 0):
            dst = lax.rem(my + off, N_DEV)
            for s in range(M_BLK // X_ROWS):
                row_jobs.append((off, dst, s))

        def x_dma(idx):
            _, dst, s = row_jobs[idx]
            return pltpu.make_async_copy(
                x_hbm.at[pl.ds(dst * M_BLK + s * X_ROWS, X_ROWS), :],
                xstage.at[idx % 2],
                x_sems.at[idx % 2],
            )

        sub_per_blk = M_BLK // X_ROWS
        a2a_sends = []
        x_dma(0).start()
        for idx in range(len(row_jobs)):
            if idx + 1 < len(row_jobs):
                x_dma(idx + 1).start()
            x_dma(idx).wait()
            off, dst, s = row_jobs[idx]
            xb[pl.ds(dst * M_BLK + s * X_ROWS, X_ROWS), :] = (
                xstage[idx % 2].astype(jnp.bfloat16))
            if s == sub_per_blk - 1:
                if off == 0:
                    xg[:, pl.ds(my * K_BLK, K_BLK)] = (
                        xb[pl.ds(my * M_BLK, M_BLK), :])
                else:
                    rdma = pltpu.make_async_remote_copy(
                        src_ref=xb.at[pl.ds(dst * M_BLK, M_BLK), :],
                        dst_ref=xg.at[:, pl.ds(my * K_BLK, K_BLK)],
                        send_sem=send_sems.at[off - 1],
                        recv_sem=recv_sems.at[my],
                        device_id=(dst,),
                        device_id_type=pl.DeviceIdType.MESH,
                    )
                    rdma.start()
                    a2a_sends.append(rdma)

        def recv_desc(src):
            return pltpu.make_async_remote_copy(
                src_ref=xb.at[pl.ds(0, M_BLK), :],
                dst_ref=xg.at[:, pl.ds(src * K_BLK, K_BLK)],
                send_sem=send_sems.at[0],
                recv_sem=recv_sems.at[src],
                device_id=(src,),
                device_id_type=pl.DeviceIdType.MESH,
            )

        block_order = [my, lax.rem(my + 1, N_DEV),
                       lax.rem(my + 3, N_DEV), lax.rem(my + 2, N_DEV)]
        k_offs = []
        for j in block_order:
            for s in range(K_BLK // K_CHUNK):
                k_offs.append(j * K_BLK + s * K_CHUNK)
        n_k = len(k_offs)
        n_t = N // N_TILE

        def w_dma(t):
            k_idx, nt = t // n_t, t % n_t
            return pltpu.make_async_copy(
                w_hbm.at[pl.ds(k_offs[k_idx], K_CHUNK),
                         pl.ds(nt * N_TILE, N_TILE)],
                wstage.at[t % 2],
                w_sems.at[t % 2],
            )

        w_dma(0).start()
        for t in range(n_k * n_t):
            if t + 1 < n_k * n_t:
                w_dma(t + 1).start()
            k_idx, nt = t // n_t, t % n_t
            blk = k_idx // (K_BLK // K_CHUNK)
            if nt == 0 and k_idx % (K_BLK // K_CHUNK) == 0 and blk > 0:
                recv_desc(block_order[blk]).wait_recv()
            w_dma(t).wait()
            xg_k = xg[:, pl.ds(k_offs[k_idx], K_CHUNK)]
            wb = wstage[t % 2].astype(jnp.bfloat16)
            part = jnp.dot(xg_k, wb, preferred_element_type=jnp.float32)
            if k_idx == 0:
                out_ref[:, pl.ds(nt * N_TILE, N_TILE)] = part
            else:
                out_ref[:, pl.ds(nt * N_TILE, N_TILE)] += part

        local_amax = jnp.float32(0.0)
        for nt in range(n_t):
            local_amax = jnp.maximum(
                local_amax,
                jnp.max(jnp.abs(out_ref[:, pl.ds(nt * N_TILE, N_TILE)])))
        amax_mine[...] = jnp.full((8, 128), local_amax, jnp.float32)
        am_sends = []
        for off in (1, 2, 3):
            dst = lax.rem(my + off, N_DEV)
            rdma = pltpu.make_async_remote_copy(
                src_ref=amax_mine,
                dst_ref=amax_buf.at[my],
                send_sem=am_send_sems.at[off - 1],
                recv_sem=am_recv_sems.at[my],
                device_id=(dst,),
                device_id_type=pl.DeviceIdType.MESH,
            )
            rdma.start()
            am_sends.append(rdma)
        for off in (1, 2, 3):
            src = lax.rem(my + off, N_DEV)
            pltpu.make_async_remote_copy(
                src_ref=amax_mine,
                dst_ref=amax_buf.at[src],
                send_sem=am_send_sems.at[0],
                recv_sem=am_recv_sems.at[src],
                device_id=(src,),
                device_id_type=pl.DeviceIdType.MESH,
            ).wait_recv()
        amax = jnp.maximum(jnp.max(amax_buf[...]), local_amax)

        scale = amax / 127.0
        inv_scale = 127.0 / amax
        for nt in range(n_t):
            y = out_ref[:, pl.ds(nt * N_TILE, N_TILE)]
            q = jnp.clip(jnp.round(y * inv_scale), -127.0, 127.0)
            out_ref[:, pl.ds(nt * N_TILE, N_TILE)] = q * scale

        for r in a2a_sends + am_sends:
            r.wait_send()

    return pl.pallas_call(
        body,
        out_shape=jax.ShapeDtypeStruct((M_BLK, N), jnp.float32),
        in_specs=[
            pl.BlockSpec(memory_space=pl.ANY),
            pl.BlockSpec(memory_space=pl.ANY),
        ],
        out_specs=pl.BlockSpec(memory_space=pltpu.VMEM),
        scratch_shapes=[
            pltpu.VMEM((N_DEV * M_BLK, K_BLK), jnp.bfloat16),
            pltpu.VMEM((M_BLK, K), jnp.bfloat16),
            pltpu.VMEM((2, X_ROWS, K_BLK), jnp.float32),
            pltpu.VMEM((2, K_CHUNK, N_TILE), jnp.float32),
            pltpu.VMEM((8, 128), jnp.float32),
            pltpu.VMEM((N_DEV, 8, 128), jnp.float32),
            pltpu.SemaphoreType.DMA((3,)),
            pltpu.SemaphoreType.DMA((N_DEV,)),
            pltpu.SemaphoreType.DMA((3,)),
            pltpu.SemaphoreType.DMA((N_DEV,)),
            pltpu.SemaphoreType.DMA((2,)),
            pltpu.SemaphoreType.DMA((2,)),
        ],
        compiler_params=pltpu.CompilerParams(
            collective_id=0,
            vmem_limit_bytes=63 * 1024 * 1024,
        ),
    )(x, w_mat)


# device time: 169963 ns/iter; 1.1575x vs baseline; 1.1575x over previous
import jax
import jax.numpy as jnp
from jax import lax
from jax.experimental import pallas as pl
from jax.experimental.pallas import tpu as pltpu

N_DEV = 4
M_BLK = 1024
K = 4096
N = 8192
K_BLK = 1024
K_CHUNK = 1024
N_TILE = 512
X_ROWS = 256


def kernel(x, w_mat):
    def body(x_hbm, w_hbm, out_ref, xb, xg, xstage, wstage, amax_mine,
             amax_buf, send_sems, recv_sems, am_send_sems, am_recv_sems,
             x_sems, w_sems):
        my = lax.axis_index("i")

        barrier = pltpu.get_barrier_semaphore()
        for off in (1, 2, 3):
            dst = lax.rem(my + off, N_DEV)
            pl.semaphore_signal(
                barrier, inc=1,
                device_id=(dst,), device_id_type=pl.DeviceIdType.MESH,
            )
        pl.semaphore_wait(barrier, N_DEV - 1)

        amax_buf[...] = jnp.zeros_like(amax_buf)

        row_jobs = []
        for off in (2, 1, 3, 0):
            dst = lax.rem(my + off, N_DEV)
            for s in range(M_BLK // X_ROWS):
                row_jobs.append((off, dst, s))

        def x_dma(idx):
            _, dst, s = row_jobs[idx]
            return pltpu.make_async_copy(
                x_hbm.at[pl.ds(dst * M_BLK + s * X_ROWS, X_ROWS), :],
                xstage.at[idx % 2],
                x_sems.at[idx % 2],
            )

        sub_per_blk = M_BLK // X_ROWS
        a2a_sends = []
        x_dma(0).start()
        for idx in range(len(row_jobs)):
            if idx + 1 < len(row_jobs):
                x_dma(idx + 1).start()
            x_dma(idx).wait()
            off, dst, s = row_jobs[idx]
            xb[pl.ds(dst * M_BLK + s * X_ROWS, X_ROWS), :] = (
                xstage[idx % 2].astype(jnp.bfloat16))
            if s == sub_per_blk - 1:
                if off == 0:
                    xg[:, pl.ds(my * K_BLK, K_BLK)] = (
                        xb[pl.ds(my * M_BLK, M_BLK), :])
                else:
                    rdma = pltpu.make_async_remote_copy(
                        src_ref=xb.at[pl.ds(dst * M_BLK, M_BLK), :],
                        dst_ref=xg.at[:, pl.ds(my * K_BLK, K_BLK)],
                        send_sem=send_sems.at[off - 1],
                        recv_sem=recv_sems.at[my],
                        device_id=(dst,),
                        device_id_type=pl.DeviceIdType.MESH,
                    )
                    rdma.start()
                    a2a_sends.append(rdma)

        def recv_desc(src):
            return pltpu.make_async_remote_copy(
                src_ref=xb.at[pl.ds(0, M_BLK), :],
                dst_ref=xg.at[:, pl.ds(src * K_BLK, K_BLK)],
                send_sem=send_sems.at[0],
                recv_sem=recv_sems.at[src],
                device_id=(src,),
                device_id_type=pl.DeviceIdType.MESH,
            )

        block_order = [my, lax.rem(my + 1, N_DEV),
                       lax.rem(my + 3, N_DEV), lax.rem(my + 2, N_DEV)]
        n_k = K // K_CHUNK
        n_t = N // N_TILE
        local_amax = jnp.float32(0.0)

        def w_dma(t):
            k_idx, nt = t // n_t, t % n_t
            return pltpu.make_async_copy(
                w_hbm.at[pl.ds(block_order[k_idx] * K_BLK, K_CHUNK),
                         pl.ds(nt * N_TILE, N_TILE)],
                wstage.at[t % 2],
                w_sems.at[t % 2],
            )

        w_dma(0).start()
        for t in range(n_k * n_t):
            if t + 1 < n_k * n_t:
                w_dma(t + 1).start()
            k_idx, nt = t // n_t, t % n_t
            if nt == 0 and k_idx > 0:
                recv_desc(block_order[k_idx]).wait_recv()
            w_dma(t).wait()
            xg_k = xg[:, pl.ds(block_order[k_idx] * K_BLK, K_CHUNK)]
            wb = wstage[t % 2].astype(jnp.bfloat16)
            part = jnp.dot(xg_k, wb, preferred_element_type=jnp.float32)
            nts = pl.ds(nt * N_TILE, N_TILE)
            if k_idx == 0:
                out_ref[:, nts] = part
            elif k_idx < n_k - 1:
                out_ref[:, nts] += part
            else:
                acc = out_ref[:, nts] + part
                out_ref[:, nts] = acc
                local_amax = jnp.maximum(local_amax, jnp.max(jnp.abs(acc)))

        amax_mine[...] = jnp.full((8, 128), local_amax, jnp.float32)
        am_sends = []
        for off in (1, 2, 3):
            dst = lax.rem(my + off, N_DEV)
            rdma = pltpu.make_async_remote_copy(
                src_ref=amax_mine,
                dst_ref=amax_buf.at[my],
                send_sem=am_send_sems.at[off - 1],
                recv_sem=am_recv_sems.at[my],
                device_id=(dst,),
                device_id_type=pl.DeviceIdType.MESH,
            )
            rdma.start()
            am_sends.append(rdma)
        for off in (1, 2, 3):
            src = lax.rem(my + off, N_DEV)
            pltpu.make_async_remote_copy(
                src_ref=amax_mine,
                dst_ref=amax_buf.at[src],
                send_sem=am_send_sems.at[0],
                recv_sem=am_recv_sems.at[src],
                device_id=(src,),
                device_id_type=pl.DeviceIdType.MESH,
            ).wait_recv()
        amax = jnp.maximum(jnp.max(amax_buf[...]), local_amax)

        scale = amax / 127.0
        inv_scale = 127.0 / amax
        for nt in range(n_t):
            y = out_ref[:, pl.ds(nt * N_TILE, N_TILE)]
            q = jnp.clip(jnp.round(y * inv_scale), -127.0, 127.0)
            out_ref[:, pl.ds(nt * N_TILE, N_TILE)] = q * scale

        for r in a2a_sends + am_sends:
            r.wait_send()

    return pl.pallas_call(
        body,
        out_shape=jax.ShapeDtypeStruct((M_BLK, N), jnp.float32),
        in_specs=[
            pl.BlockSpec(memory_space=pl.ANY),
            pl.BlockSpec(memory_space=pl.ANY),
        ],
        out_specs=pl.BlockSpec(memory_space=pltpu.VMEM),
        scratch_shapes=[
            pltpu.VMEM((N_DEV * M_BLK, K_BLK), jnp.bfloat16),
            pltpu.VMEM((M_BLK, K), jnp.bfloat16),
            pltpu.VMEM((2, X_ROWS, K_BLK), jnp.float32),
            pltpu.VMEM((2, K_CHUNK, N_TILE), jnp.float32),
            pltpu.VMEM((8, 128), jnp.float32),
            pltpu.VMEM((N_DEV, 8, 128), jnp.float32),
            pltpu.SemaphoreType.DMA((3,)),
            pltpu.SemaphoreType.DMA((N_DEV,)),
            pltpu.SemaphoreType.DMA((3,)),
            pltpu.SemaphoreType.DMA((N_DEV,)),
            pltpu.SemaphoreType.DMA((2,)),
            pltpu.SemaphoreType.DMA((2,)),
        ],
        compiler_params=pltpu.CompilerParams(
            collective_id=0,
            vmem_limit_bytes=63 * 1024 * 1024,
        ),
    )(x, w_mat)


# device time: 149155 ns/iter; 1.3190x vs baseline; 1.1395x over previous
import os

import jax
import jax.numpy as jnp
from jax import lax
from jax.experimental import pallas as pl
from jax.experimental.pallas import tpu as pltpu

_ABL = os.environ.get("KERNEL_ABL", "full")

N_DEV = 4
M_BLK = 1024
K = 4096
N = 8192
K_BLK = 1024
K_CHUNK = 1024
N_TILE = 512
X_ROWS = 256


def kernel(x, w_mat):
    def body(x_hbm, w_hbm, out_ref, xb, xg, xstage, wstage, amax_mine,
             amax_buf, send_sems, recv_sems, am_send_sems, am_recv_sems,
             x_sems, w_sems):
        my = lax.axis_index("i")

        barrier = pltpu.get_barrier_semaphore()
        for off in (1, 2, 3):
            dst = lax.rem(my + off, N_DEV)
            pl.semaphore_signal(
                barrier, inc=1,
                device_id=(dst,), device_id_type=pl.DeviceIdType.MESH,
            )
        pl.semaphore_wait(barrier, N_DEV - 1)

        amax_buf[...] = jnp.zeros_like(amax_buf)

        row_jobs = []
        for off in (2, 1, 3, 0):
            dst = lax.rem(my + off, N_DEV)
            for s in range(M_BLK // X_ROWS):
                row_jobs.append((off, dst, s))

        def x_dma(idx):
            _, dst, s = row_jobs[idx]
            return pltpu.make_async_copy(
                x_hbm.at[pl.ds(dst * M_BLK + s * X_ROWS, X_ROWS), :],
                xstage.at[idx % 2],
                x_sems.at[idx % 2],
            )

        sub_per_blk = M_BLK // X_ROWS
        a2a_sends = []
        x_dma(0).start()
        for idx in range(len(row_jobs)):
            if idx + 1 < len(row_jobs):
                x_dma(idx + 1).start()
            x_dma(idx).wait()
            off, dst, s = row_jobs[idx]
            xb[pl.ds(dst * M_BLK + s * X_ROWS, X_ROWS), :] = (
                xstage[idx % 2].astype(jnp.bfloat16))
            if s == sub_per_blk - 1:
                if off == 0:
                    xg[:, pl.ds(my * K_BLK, K_BLK)] = (
                        xb[pl.ds(my * M_BLK, M_BLK), :])
                elif _ABL == "noa2a":
                    pass
                else:
                    rdma = pltpu.make_async_remote_copy(
                        src_ref=xb.at[pl.ds(dst * M_BLK, M_BLK), :],
                        dst_ref=xg.at[:, pl.ds(my * K_BLK, K_BLK)],
                        send_sem=send_sems.at[off - 1],
                        recv_sem=recv_sems.at[my],
                        device_id=(dst,),
                        device_id_type=pl.DeviceIdType.MESH,
                    )
                    rdma.start()
                    a2a_sends.append(rdma)

        def recv_desc(src):
            return pltpu.make_async_remote_copy(
                src_ref=xb.at[pl.ds(0, M_BLK), :],
                dst_ref=xg.at[:, pl.ds(src * K_BLK, K_BLK)],
                send_sem=send_sems.at[0],
                recv_sem=recv_sems.at[src],
                device_id=(src,),
                device_id_type=pl.DeviceIdType.MESH,
            )

        block_order = [my, lax.rem(my + 1, N_DEV),
                       lax.rem(my + 3, N_DEV), lax.rem(my + 2, N_DEV)]
        if _ABL == "noa2a":
            block_order = [my, my, my, my]
        n_k = K // K_CHUNK
        n_t = N // N_TILE
        local_amax = jnp.float32(0.0)

        def w_dma(t):
            k_idx, nt = t // n_t, t % n_t
            return pltpu.make_async_copy(
                w_hbm.at[pl.ds(block_order[k_idx] * K_BLK, K_CHUNK),
                         pl.ds(nt * N_TILE, N_TILE)],
                wstage.at[t % 2],
                w_sems.at[t % 2],
            )

        if _ABL == "nogemm":
            for nt in range(n_t):
                out_ref[:, pl.ds(nt * N_TILE, N_TILE)] = jnp.zeros(
                    (M_BLK, N_TILE), jnp.float32)
            for off in (1, 2, 3):
                recv_desc(lax.rem(my + off, N_DEV)).wait_recv()
        if _ABL != "nogemm":
            w_dma(0).start()
        for t in range(n_k * n_t if _ABL != "nogemm" else 0):
            if t + 1 < n_k * n_t:
                w_dma(t + 1).start()
            k_idx, nt = t // n_t, t % n_t
            if nt == 0 and k_idx > 0 and _ABL != "noa2a":
                recv_desc(block_order[k_idx]).wait_recv()
            w_dma(t).wait()
            xg_k = xg[:, pl.ds(block_order[k_idx] * K_BLK, K_CHUNK)]
            wb = wstage[t % 2].astype(jnp.bfloat16)
            part = jnp.dot(xg_k, wb, preferred_element_type=jnp.float32)
            nts = pl.ds(nt * N_TILE, N_TILE)
            if k_idx == 0:
                out_ref[:, nts] = part
            elif k_idx < n_k - 1:
                out_ref[:, nts] += part
            else:
                acc = out_ref[:, nts] + part
                out_ref[:, nts] = acc
                local_amax = jnp.maximum(local_amax, jnp.max(jnp.abs(acc)))

        amax_mine[...] = jnp.full((8, 128), local_amax, jnp.float32)
        am_sends = []
        do_exchange = _ABL in ("full", "nogemm")
        for off in (1, 2, 3) if do_exchange else ():
            dst = lax.rem(my + off, N_DEV)
            rdma = pltpu.make_async_remote_copy(
                src_ref=amax_mine,
                dst_ref=amax_buf.at[my],
                send_sem=am_send_sems.at[off - 1],
                recv_sem=am_recv_sems.at[my],
                device_id=(dst,),
                device_id_type=pl.DeviceIdType.MESH,
            )
            rdma.start()
            am_sends.append(rdma)
        for off in (1, 2, 3) if do_exchange else ():
            src = lax.rem(my + off, N_DEV)
            pltpu.make_async_remote_copy(
                src_ref=amax_mine,
                dst_ref=amax_buf.at[src],
                send_sem=am_send_sems.at[0],
                recv_sem=am_recv_sems.at[src],
                device_id=(src,),
                device_id_type=pl.DeviceIdType.MESH,
            ).wait_recv()
        amax = jnp.maximum(jnp.max(amax_buf[...]), local_amax)

        scale = amax / 127.0
        inv_scale = 127.0 / amax
        for nt in range(n_t if _ABL != "noquant" else 0):
            y = out_ref[:, pl.ds(nt * N_TILE, N_TILE)]
            q = jnp.clip(jnp.round(y * inv_scale), -127.0, 127.0)
            out_ref[:, pl.ds(nt * N_TILE, N_TILE)] = q * scale

        for r in a2a_sends + am_sends:
            r.wait_send()

    return pl.pallas_call(
        body,
        out_shape=jax.ShapeDtypeStruct((M_BLK, N), jnp.float32),
        in_specs=[
            pl.BlockSpec(memory_space=pl.ANY),
            pl.BlockSpec(memory_space=pl.ANY),
        ],
        out_specs=pl.BlockSpec(memory_space=pltpu.VMEM),
        scratch_shapes=[
            pltpu.VMEM((N_DEV * M_BLK, K_BLK), jnp.bfloat16),
            pltpu.VMEM((M_BLK, K), jnp.bfloat16),
            pltpu.VMEM((2, X_ROWS, K_BLK), jnp.float32),
            pltpu.VMEM((2, K_CHUNK, N_TILE), jnp.float32),
            pltpu.VMEM((8, 128), jnp.float32),
            pltpu.VMEM((N_DEV, 8, 128), jnp.float32),
            pltpu.SemaphoreType.DMA((3,)),
            pltpu.SemaphoreType.DMA((N_DEV,)),
            pltpu.SemaphoreType.DMA((3,)),
            pltpu.SemaphoreType.DMA((N_DEV,)),
            pltpu.SemaphoreType.DMA((2,)),
            pltpu.SemaphoreType.DMA((2,)),
        ],
        compiler_params=pltpu.CompilerParams(
            collective_id=0,
            vmem_limit_bytes=63 * 1024 * 1024,
        ),
    )(x, w_mat)


# device time: 124640 ns/iter; 1.5784x vs baseline; 1.1967x over previous
import os

import jax
import jax.numpy as jnp
from jax import lax
from jax.experimental import pallas as pl
from jax.experimental.pallas import tpu as pltpu

_ABL = os.environ.get("KERNEL_ABL", "full")

N_DEV = 4
M_BLK = 1024
K = 4096
N = 8192
K_BLK = 1024
K_CHUNK = 1024
N_TILE = 512
X_ROWS = 256


def kernel(x, w_mat):
    def body(x_hbm, w_hbm, out_ref, xb, xg, xstage, wstage, wdot, amax_mine,
             amax_buf, send_sems, recv_sems, am_send_sems, am_recv_sems,
             x_sems, w_sems):
        my = lax.axis_index("i")

        barrier = pltpu.get_barrier_semaphore()
        for off in (1, 2, 3):
            dst = lax.rem(my + off, N_DEV)
            pl.semaphore_signal(
                barrier, inc=1,
                device_id=(dst,), device_id_type=pl.DeviceIdType.MESH,
            )
        pl.semaphore_wait(barrier, N_DEV - 1)

        amax_buf[...] = jnp.zeros_like(amax_buf)

        row_jobs = []
        for off in (2, 1, 3, 0):
            dst = lax.rem(my + off, N_DEV)
            for s in range(M_BLK // X_ROWS):
                row_jobs.append((off, dst, s))

        def x_dma(idx):
            _, dst, s = row_jobs[idx]
            return pltpu.make_async_copy(
                x_hbm.at[pl.ds(dst * M_BLK + s * X_ROWS, X_ROWS), :],
                xstage.at[idx % 2],
                x_sems.at[idx % 2],
            )

        sub_per_blk = M_BLK // X_ROWS
        a2a_sends = []
        x_dma(0).start()
        for idx in range(len(row_jobs)):
            if idx + 1 < len(row_jobs):
                x_dma(idx + 1).start()
            x_dma(idx).wait()
            off, dst, s = row_jobs[idx]
            xb[pl.ds(dst * M_BLK + s * X_ROWS, X_ROWS), :] = (
                xstage[idx % 2].astype(jnp.bfloat16))
            if s == sub_per_blk - 1:
                if off == 0:
                    xg[:, pl.ds(my * K_BLK, K_BLK)] = (
                        xb[pl.ds(my * M_BLK, M_BLK), :])
                elif _ABL in ("noa2a", "dotonly"):
                    pass
                else:
                    rdma = pltpu.make_async_remote_copy(
                        src_ref=xb.at[pl.ds(dst * M_BLK, M_BLK), :],
                        dst_ref=xg.at[:, pl.ds(my * K_BLK, K_BLK)],
                        send_sem=send_sems.at[off - 1],
                        recv_sem=recv_sems.at[my],
                        device_id=(dst,),
                        device_id_type=pl.DeviceIdType.MESH,
                    )
                    rdma.start()
                    a2a_sends.append(rdma)

        def recv_desc(src):
            return pltpu.make_async_remote_copy(
                src_ref=xb.at[pl.ds(0, M_BLK), :],
                dst_ref=xg.at[:, pl.ds(src * K_BLK, K_BLK)],
                send_sem=send_sems.at[0],
                recv_sem=recv_sems.at[src],
                device_id=(src,),
                device_id_type=pl.DeviceIdType.MESH,
            )

        block_order = [my, lax.rem(my + 1, N_DEV),
                       lax.rem(my + 3, N_DEV), lax.rem(my + 2, N_DEV)]
        if _ABL == "noa2a":
            block_order = [my, my, my, my]
        n_k = K // K_CHUNK
        n_t = N // N_TILE
        local_amax = jnp.float32(0.0)

        def w_dma(t):
            k_idx, nt = t // n_t, t % n_t
            return pltpu.make_async_copy(
                w_hbm.at[pl.ds(block_order[k_idx] * K_BLK, K_CHUNK),
                         pl.ds(nt * N_TILE, N_TILE)],
                wstage.at[t % 2],
                w_sems.at[t % 2],
            )

        if _ABL == "dotonly":
            for nt in range(n_t):
                out_ref[:, pl.ds(nt * N_TILE, N_TILE)] = jnp.dot(
                    xg[:, :], wdot[...],
                    preferred_element_type=jnp.float32)
        if _ABL == "nogemm":
            for nt in range(n_t):
                out_ref[:, pl.ds(nt * N_TILE, N_TILE)] = jnp.zeros(
                    (M_BLK, N_TILE), jnp.float32)
            for off in (1, 2, 3):
                recv_desc(lax.rem(my + off, N_DEV)).wait_recv()
        if _ABL not in ("nogemm", "dotonly"):
            w_dma(0).start()
        for t in range(n_k * n_t if _ABL not in ("nogemm", "dotonly") else 0):
            if t + 1 < n_k * n_t:
                w_dma(t + 1).start()
            k_idx, nt = t // n_t, t % n_t
            if nt == 0 and k_idx > 0 and _ABL != "noa2a":
                recv_desc(block_order[k_idx]).wait_recv()
            w_dma(t).wait()
            xg_k = xg[:, pl.ds(block_order[k_idx] * K_BLK, K_CHUNK)]
            wb = wstage[t % 2].astype(jnp.bfloat16)
            part = jnp.dot(xg_k, wb, preferred_element_type=jnp.float32)
            nts = pl.ds(nt * N_TILE, N_TILE)
            if k_idx == 0:
                out_ref[:, nts] = part
            elif k_idx < n_k - 1:
                out_ref[:, nts] += part
            else:
                acc = out_ref[:, nts] + part
                out_ref[:, nts] = acc
                local_amax = jnp.maximum(local_amax, jnp.max(jnp.abs(acc)))

        amax_mine[...] = jnp.full((8, 128), local_amax, jnp.float32)
        am_sends = []
        do_exchange = _ABL in ("full", "nogemm")
        for off in (1, 2, 3) if do_exchange else ():
            dst = lax.rem(my + off, N_DEV)
            rdma = pltpu.make_async_remote_copy(
                src_ref=amax_mine,
                dst_ref=amax_buf.at[my],
                send_sem=am_send_sems.at[off - 1],
                recv_sem=am_recv_sems.at[my],
                device_id=(dst,),
                device_id_type=pl.DeviceIdType.MESH,
            )
            rdma.start()
            am_sends.append(rdma)
        for off in (1, 2, 3) if do_exchange else ():
            src = lax.rem(my + off, N_DEV)
            pltpu.make_async_remote_copy(
                src_ref=amax_mine,
                dst_ref=amax_buf.at[src],
                send_sem=am_send_sems.at[0],
                recv_sem=am_recv_sems.at[src],
                device_id=(src,),
                device_id_type=pl.DeviceIdType.MESH,
            ).wait_recv()
        amax = jnp.maximum(jnp.max(amax_buf[...]), local_amax)

        scale = amax / 127.0
        inv_scale = 127.0 / amax
        for nt in range(n_t if _ABL != "noquant" else 0):
            y = out_ref[:, pl.ds(nt * N_TILE, N_TILE)]
            q = jnp.clip(jnp.round(y * inv_scale), -127.0, 127.0)
            out_ref[:, pl.ds(nt * N_TILE, N_TILE)] = q * scale

        for r in a2a_sends + am_sends:
            r.wait_send()

    return pl.pallas_call(
        body,
        out_shape=jax.ShapeDtypeStruct((M_BLK, N), jnp.float32),
        in_specs=[
            pl.BlockSpec(memory_space=pl.ANY),
            pl.BlockSpec(memory_space=pl.ANY),
        ],
        out_specs=pl.BlockSpec(memory_space=pltpu.VMEM),
        scratch_shapes=[
            pltpu.VMEM((N_DEV * M_BLK, K_BLK), jnp.bfloat16),
            pltpu.VMEM((M_BLK, K), jnp.bfloat16),
            pltpu.VMEM((2, X_ROWS, K_BLK), jnp.float32),
            pltpu.VMEM((2, K_CHUNK, N_TILE), jnp.float32),
            pltpu.VMEM((K, N_TILE), jnp.bfloat16),
            pltpu.VMEM((8, 128), jnp.float32),
            pltpu.VMEM((N_DEV, 8, 128), jnp.float32),
            pltpu.SemaphoreType.DMA((3,)),
            pltpu.SemaphoreType.DMA((N_DEV,)),
            pltpu.SemaphoreType.DMA((3,)),
            pltpu.SemaphoreType.DMA((N_DEV,)),
            pltpu.SemaphoreType.DMA((2,)),
            pltpu.SemaphoreType.DMA((2,)),
        ],
        compiler_params=pltpu.CompilerParams(
            collective_id=0,
            vmem_limit_bytes=63 * 1024 * 1024,
        ),
    )(x, w_mat)
